# Initial kernel scaffold; baseline (speedup 1.0000x reference)
import jax
import jax.numpy as jnp
from jax.experimental import pallas as pl


def kernel(inp, table):
    raise NotImplementedError("write your pallas kernel here")


# speedup vs baseline: 2.7581x; 2.7581x over previous
import functools

import jax
import jax.numpy as jnp
from jax import lax
from jax.experimental import pallas as pl
from jax.experimental.pallas import tpu as pltpu
from jax.experimental.pallas import tpu_sc as plsc


@functools.lru_cache(maxsize=None)
def _make_embed_kernel(V, D, B, T):
    info = plsc.get_sparse_core_info()
    NC, NS, L = info.num_cores, info.num_subcores, info.num_lanes
    NW = NC * NS
    assert B % NW == 0 and D % L == 0
    per_w = B // NW
    CH = 16
    NCH = per_w // CH
    ROWS = CH * T
    inv_t = 1.0 / T

    mesh = plsc.VectorSubcoreMesh(core_axis_name="c", subcore_axis_name="s")

    @functools.partial(
        pl.kernel,
        mesh=mesh,
        out_type=jax.ShapeDtypeStruct((B * D,), jnp.float32),
        compiler_params=pltpu.CompilerParams(use_tc_tiling_on_sc=False),
        scratch_types=[
            pltpu.VMEM((per_w * T,), jnp.int32),
            pltpu.VMEM((ROWS, D), jnp.float32),
            pltpu.VMEM((per_w * D,), jnp.float32),
            pltpu.SemaphoreType.DMA,
        ],
    )
    def k(table_hbm, idx_hbm, out_hbm, idx_v, rows_v, out_v, sem):
        wid = lax.axis_index("s") * NC + lax.axis_index("c")
        pltpu.sync_copy(idx_hbm.at[pl.ds(wid * per_w * T, per_w * T)], idx_v)

        def chunk_body(g, carry):
            pltpu.async_copy(
                table_hbm.at[idx_v.at[pl.ds(g * ROWS, ROWS)]], rows_v, sem
            ).wait()

            def elem_body(b, carry2):
                r0 = b * T
                accs = [jnp.zeros((L,), jnp.float32) for _ in range(D // L)]
                for t in range(T):
                    for h in range(D // L):
                        accs[h] = accs[h] + rows_v[r0 + t, pl.ds(h * L, L)]
                o = (g * CH + b) * D
                for h in range(D // L):
                    out_v[pl.ds(o + h * L, L)] = accs[h] * inv_t
                return carry2

            return lax.fori_loop(0, CH, elem_body, carry)

        lax.fori_loop(0, NCH, chunk_body, 0)
        pltpu.sync_copy(out_v, out_hbm.at[pl.ds(wid * per_w * D, per_w * D)])

    return k


def kernel(inp, table):
    B, T = inp.shape
    V, D = table.shape
    k = _make_embed_kernel(V, D, B, T)
    out = k(table, inp.reshape(-1))
    return out.reshape(B, D)


# speedup vs baseline: 2.9000x; 1.0515x over previous
import functools

import jax
import jax.numpy as jnp
from jax import lax
from jax.experimental import pallas as pl
from jax.experimental.pallas import tpu as pltpu
from jax.experimental.pallas import tpu_sc as plsc


@functools.lru_cache(maxsize=None)
def _make_embed_kernel(V, D, B, T):
    info = plsc.get_sparse_core_info()
    NC, NS, L = info.num_cores, info.num_subcores, info.num_lanes
    NW = NC * NS
    assert B % NW == 0 and D % L == 0
    per_w = B // NW
    CH = 16
    NCH = per_w // CH
    ROWS = CH * T
    inv_t = 1.0 / T

    mesh = plsc.VectorSubcoreMesh(core_axis_name="c", subcore_axis_name="s")

    @functools.partial(
        pl.kernel,
        mesh=mesh,
        out_type=jax.ShapeDtypeStruct((B * D,), jnp.float32),
        compiler_params=pltpu.CompilerParams(use_tc_tiling_on_sc=False),
        scratch_types=[
            pltpu.VMEM((per_w * T,), jnp.int32),
            pltpu.VMEM((ROWS, D), jnp.float32),
            pltpu.VMEM((ROWS, D), jnp.float32),
            pltpu.VMEM((per_w * D,), jnp.float32),
            pltpu.SemaphoreType.DMA,
            pltpu.SemaphoreType.DMA,
        ],
    )
    def k(table_hbm, idx_hbm, out_hbm, idx_v, rows0, rows1, out_v, sem0, sem1):
        wid = lax.axis_index("s") * NC + lax.axis_index("c")
        pltpu.sync_copy(idx_hbm.at[pl.ds(wid * per_w * T, per_w * T)], idx_v)

        bufs = (rows0, rows1)
        sems = (sem0, sem1)

        def start(g):
            return pltpu.async_copy(
                table_hbm.at[idx_v.at[pl.ds(g * ROWS, ROWS)]],
                bufs[g % 2],
                sems[g % 2],
            )

        def reduce_chunk(g, rows_v):
            def elem_body(b, carry2):
                r0 = b * T
                accs = [jnp.zeros((L,), jnp.float32) for _ in range(D // L)]
                for t in range(T):
                    for h in range(D // L):
                        accs[h] = accs[h] + rows_v[r0 + t, pl.ds(h * L, L)]
                o = (g * CH + b) * D
                for h in range(D // L):
                    out_v[pl.ds(o + h * L, L)] = accs[h] * inv_t
                return carry2

            lax.fori_loop(0, CH, elem_body, 0)

        pending = start(0)
        for g in range(NCH):
            nxt = start(g + 1) if g + 1 < NCH else None
            pending.wait()
            reduce_chunk(g, bufs[g % 2])
            pending = nxt

        pltpu.sync_copy(out_v, out_hbm.at[pl.ds(wid * per_w * D, per_w * D)])

    return k


def kernel(inp, table):
    B, T = inp.shape
    V, D = table.shape
    k = _make_embed_kernel(V, D, B, T)
    out = k(table, inp.reshape(-1))
    return out.reshape(B, D)
